# Initial kernel scaffold; baseline (speedup 1.0000x reference)
import jax
import jax.numpy as jnp
from jax.experimental import pallas as pl


def kernel(x, edge_index, W, b):
    raise NotImplementedError("write your pallas kernel here")


# speedup vs baseline: 6.5402x; 6.5402x over previous
import functools

import jax
import jax.numpy as jnp
from jax import lax
from jax.experimental import pallas as pl
from jax.experimental.pallas import tpu as pltpu
from jax.experimental.pallas import tpu_sc as plsc

_NC = 2
_NS = 16
_NW = _NC * _NS
_LANES = 16
_CHUNK = 128


def _sc_aggregate(x, src, dst, n_pad):
  d = x.shape[1]
  n_edges = src.shape[0]
  assert n_edges % _CHUNK == 0
  n_chunks = n_edges // _CHUNK
  rows_per_tile = n_pad // _NS
  assert n_pad % _NS == 0 and rows_per_tile % 8 == 0
  zr = 128
  assert rows_per_tile % zr == 0

  mesh = plsc.VectorSubcoreMesh(core_axis_name="c", subcore_axis_name="s")

  @functools.partial(
      pl.kernel,
      mesh=mesh,
      out_type=jax.ShapeDtypeStruct((_NC, n_pad, d), jnp.float32),
      scratch_types=[
          pltpu.VMEM_SHARED((n_pad, d), jnp.float32),
          pltpu.VMEM((_CHUNK,), jnp.int32),
          pltpu.VMEM((_CHUNK,), jnp.int32),
          pltpu.VMEM((_CHUNK, d), jnp.float32),
          pltpu.VMEM((zr, d), jnp.float32),
          pltpu.SemaphoreType.DMA,
      ],
  )
  def agg_kernel(x_hbm, src_hbm, dst_hbm, out_hbm, acc, src_v, dst_v, rows_v,
                 zbuf, sem):
    c = lax.axis_index("c")
    s = lax.axis_index("s")
    wid = s * _NC + c

    zero = jnp.zeros((_LANES,), jnp.float32)

    def zrow(r, carry):
      def zcol(j, carry2):
        zbuf[r, pl.ds(j * _LANES, _LANES)] = zero
        return carry2

      return lax.fori_loop(0, d // _LANES, zcol, carry)

    lax.fori_loop(0, zr, zrow, 0)

    row0 = s * rows_per_tile

    def zcopy(t, carry):
      pltpu.sync_copy(zbuf, acc.at[pl.ds(row0 + t * zr, zr)])
      return carry

    lax.fori_loop(0, rows_per_tile // zr, zcopy, 0)
    plsc.subcore_barrier()

    n_mine = (n_chunks - wid + _NW - 1) // _NW

    def body(i, carry):
      base = (wid + i * _NW) * _CHUNK
      pltpu.sync_copy(src_hbm.at[pl.ds(base, _CHUNK)], src_v)
      pltpu.sync_copy(dst_hbm.at[pl.ds(base, _CHUNK)], dst_v)
      pltpu.async_copy(x_hbm.at[src_v], rows_v, sem).wait()
      pltpu.sync_copy(rows_v, acc.at[dst_v], add=True)
      return carry

    lax.fori_loop(0, n_mine, body, 0)
    plsc.subcore_barrier()

    pltpu.sync_copy(
        acc.at[pl.ds(row0, rows_per_tile)],
        out_hbm.at[c, pl.ds(row0, rows_per_tile)],
    )

  return agg_kernel(x, src, dst)


def _tc_linear_relu(p0, p1, w_t, b2):
  n_nodes, d = p0.shape
  blk = 1024
  assert n_nodes % blk == 0

  def body(p0_ref, p1_ref, wt_ref, b_ref, o_ref):
    agg = p0_ref[...] + p1_ref[...]
    y = jnp.dot(agg, wt_ref[...], preferred_element_type=jnp.float32)
    o_ref[...] = jnp.maximum(y + b_ref[...], 0.0)

  return pl.pallas_call(
      body,
      grid=(n_nodes // blk,),
      in_specs=[
          pl.BlockSpec((blk, d), lambda i: (i, 0)),
          pl.BlockSpec((blk, d), lambda i: (i, 0)),
          pl.BlockSpec((d, d), lambda i: (0, 0)),
          pl.BlockSpec((1, d), lambda i: (0, 0)),
      ],
      out_specs=pl.BlockSpec((blk, d), lambda i: (i, 0)),
      out_shape=jax.ShapeDtypeStruct((n_nodes, d), jnp.float32),
  )(p0, p1, w_t, b2)


def kernel(x, edge_index, W, b):
  src = edge_index[0].astype(jnp.int32)
  dst = edge_index[1].astype(jnp.int32)
  n_nodes = x.shape[0]
  n_pad = ((n_nodes + _NS * 8 - 1) // (_NS * 8)) * _NS * 8
  n_pad = max(n_pad, 1024)
  while n_pad % 1024 != 0:
    n_pad += _NS * 8
  partials = _sc_aggregate(x, src, dst, n_pad)
  h = _tc_linear_relu(partials[0], partials[1], W.T, b.reshape(1, -1))
  return h[:n_nodes]


# speedup vs baseline: 12.2916x; 1.8794x over previous
import functools

import jax
import jax.numpy as jnp
from jax import lax
from jax.experimental import pallas as pl
from jax.experimental.pallas import tpu as pltpu
from jax.experimental.pallas import tpu_sc as plsc

_NC = 2
_NS = 16
_NW = _NC * _NS
_LANES = 16
_CHUNK = 128
_IBLK = 8


def _sc_aggregate(x, src2, dst2, n_pad):
  d = x.shape[1]
  n_chunks = src2.shape[0]
  assert n_chunks % (_NW * _IBLK) == 0
  cpw = n_chunks // _NW
  nblk = cpw // _IBLK
  assert cpw % 2 == 0 and nblk >= 2
  rows_per_tile = n_pad // _NS
  assert n_pad % _NS == 0 and rows_per_tile % 8 == 0
  zr = 64
  assert rows_per_tile % zr == 0 and d * 4 >= zr

  mesh = plsc.VectorSubcoreMesh(core_axis_name="c", subcore_axis_name="s")

  @functools.partial(
      pl.kernel,
      mesh=mesh,
      out_type=jax.ShapeDtypeStruct((_NC, n_pad, d), jnp.float32),
      scratch_types=[
          pltpu.VMEM_SHARED((n_pad, d), jnp.float32),
          pltpu.VMEM((2 * _IBLK, _CHUNK), jnp.int32),
          pltpu.VMEM((2 * _IBLK, _CHUNK), jnp.int32),
          pltpu.VMEM((_CHUNK, d), jnp.float32),
          pltpu.VMEM((_CHUNK, d), jnp.float32),
          pltpu.SemaphoreType.DMA,
          pltpu.SemaphoreType.DMA,
          pltpu.SemaphoreType.DMA,
      ],
  )
  def agg_kernel(x_hbm, src_hbm, dst_hbm, out_hbm, acc, src_ib, dst_ib,
                 rows0, rows1, isem, gsem0, gsem1):
    c = lax.axis_index("c")
    s = lax.axis_index("s")
    wid = s * _NC + c
    blk0 = wid * nblk

    def start_idx(b):
      slot = pl.multiple_of((b % 2) * _IBLK, _IBLK)
      hb = pl.multiple_of((blk0 + b) * _IBLK, _IBLK)
      pltpu.async_copy(src_hbm.at[pl.ds(hb, _IBLK)],
                       src_ib.at[pl.ds(slot, _IBLK)], isem)
      pltpu.async_copy(dst_hbm.at[pl.ds(hb, _IBLK)],
                       dst_ib.at[pl.ds(slot, _IBLK)], isem)

    def wait_idx(b):
      slot = pl.multiple_of((b % 2) * _IBLK, _IBLK)
      hb = pl.multiple_of((blk0 + b) * _IBLK, _IBLK)
      pltpu.make_async_copy(src_hbm.at[pl.ds(hb, _IBLK)],
                            src_ib.at[pl.ds(slot, _IBLK)], isem).wait()
      pltpu.make_async_copy(dst_hbm.at[pl.ds(hb, _IBLK)],
                            dst_ib.at[pl.ds(slot, _IBLK)], isem).wait()

    start_idx(0)

    zero = jnp.zeros((_LANES,), jnp.float32)

    def zrow(r, carry):
      def zcol(j, carry2):
        rows0[r, pl.ds(j * _LANES, _LANES)] = zero
        return carry2

      return lax.fori_loop(0, d // _LANES, zcol, carry)

    lax.fori_loop(0, zr, zrow, 0)

    row0 = s * rows_per_tile

    def zcopy(t, carry):
      pltpu.sync_copy(rows0.at[pl.ds(0, zr)], acc.at[pl.ds(row0 + t * zr, zr)])
      return carry

    lax.fori_loop(0, rows_per_tile // zr, zcopy, 0)
    wait_idx(0)
    start_idx(1)
    plsc.subcore_barrier()

    rows = (rows0, rows1)
    gsems = (gsem0, gsem1)

    def start_gather(i, j):
      pltpu.async_copy(x_hbm.at[src_ib.at[i % (2 * _IBLK)]], rows[j], gsems[j])

    def wait_gather(i, j):
      pltpu.make_async_copy(x_hbm.at[src_ib.at[i % (2 * _IBLK)]], rows[j],
                            gsems[j]).wait()

    start_gather(0, 0)

    def body(t, carry):
      for j in (0, 1):
        i = t * 2 + j

        @pl.when(i + 1 < cpw)
        def _():
          start_gather(i + 1, 1 - j)

        wait_gather(i, j)
        pltpu.sync_copy(rows[j], acc.at[dst_ib.at[i % (2 * _IBLK)]], add=True)

        b = i // _IBLK
        r = i % _IBLK

        @pl.when((r == _IBLK - 2) & (b + 1 < nblk))
        def _():
          wait_idx(b + 1)

        @pl.when((r == _IBLK - 1) & (b + 2 < nblk))
        def _():
          start_idx(b + 2)

      return carry

    lax.fori_loop(0, cpw // 2, body, 0)
    plsc.subcore_barrier()

    pltpu.sync_copy(
        acc.at[pl.ds(row0, rows_per_tile)],
        out_hbm.at[c, pl.ds(row0, rows_per_tile)],
    )

  return agg_kernel(x, src2, dst2)


def _tc_linear_relu(p0, p1, w_t, b2):
  n_nodes, d = p0.shape
  blk = 1024
  assert n_nodes % blk == 0

  def body(p0_ref, p1_ref, wt_ref, b_ref, o_ref):
    agg = p0_ref[...] + p1_ref[...]
    y = jnp.dot(agg, wt_ref[...], preferred_element_type=jnp.float32)
    o_ref[...] = jnp.maximum(y + b_ref[...], 0.0)

  return pl.pallas_call(
      body,
      grid=(n_nodes // blk,),
      in_specs=[
          pl.BlockSpec((blk, d), lambda i: (i, 0)),
          pl.BlockSpec((blk, d), lambda i: (i, 0)),
          pl.BlockSpec((d, d), lambda i: (0, 0)),
          pl.BlockSpec((1, d), lambda i: (0, 0)),
      ],
      out_specs=pl.BlockSpec((blk, d), lambda i: (i, 0)),
      out_shape=jax.ShapeDtypeStruct((n_nodes, d), jnp.float32),
  )(p0, p1, w_t, b2)


def kernel(x, edge_index, W, b):
  src = edge_index[0].astype(jnp.int32)
  dst = edge_index[1].astype(jnp.int32)
  n_nodes = x.shape[0]
  n_edges = src.shape[0]

  n_pad = n_nodes
  while n_pad % 1024 != 0 or (n_pad // _NS) % 8 != 0:
    n_pad += 1
  pad_rows = n_pad - n_nodes

  unit = _CHUNK * _NW * _IBLK
  e_pad = ((n_edges + unit - 1) // unit) * unit
  extra = e_pad - n_edges
  if extra and pad_rows == 0:
    n_pad += 1
    while n_pad % 1024 != 0 or (n_pad // _NS) % 8 != 0:
      n_pad += 1
    pad_rows = n_pad - n_nodes
  if extra:
    fill = jnp.arange(extra, dtype=jnp.int32)
    src = jnp.concatenate([src, fill % n_nodes])
    dst = jnp.concatenate([dst, n_nodes + fill % max(pad_rows, 1)])
  src2 = src.reshape(-1, _CHUNK)
  dst2 = dst.reshape(-1, _CHUNK)

  partials = _sc_aggregate(x, src2, dst2, n_pad)
  h = _tc_linear_relu(partials[0], partials[1], W.T, b.reshape(1, -1))
  return h[:n_nodes]


# speedup vs baseline: 13.3245x; 1.0840x over previous
import functools

import jax
import jax.numpy as jnp
from jax import lax
from jax.experimental import pallas as pl
from jax.experimental.pallas import tpu as pltpu
from jax.experimental.pallas import tpu_sc as plsc

_NC = 2
_NS = 16
_NW = _NC * _NS
_LANES = 16
_CHUNK = 64
_IBLK = 8


def _sc_aggregate(x, src2, dst2, n_pad):
  d = x.shape[1]
  n_chunks = src2.shape[0]
  assert n_chunks % (_NW * _IBLK) == 0
  cpw = n_chunks // _NW
  nblk = cpw // _IBLK
  assert cpw % 4 == 0 and nblk >= 2
  rows_per_tile = n_pad // _NS
  assert n_pad % _NS == 0 and rows_per_tile % 8 == 0
  zr = 64
  assert rows_per_tile % zr == 0 and d * 4 >= zr

  mesh = plsc.VectorSubcoreMesh(core_axis_name="c", subcore_axis_name="s")

  @functools.partial(
      pl.kernel,
      mesh=mesh,
      out_type=jax.ShapeDtypeStruct((_NC, n_pad, d), jnp.float32),
      scratch_types=[
          pltpu.VMEM_SHARED((n_pad, d), jnp.float32),
          pltpu.VMEM((2 * _IBLK, _CHUNK), jnp.int32),
          pltpu.VMEM((2 * _IBLK, _CHUNK), jnp.int32),
          pltpu.VMEM((_CHUNK, d), jnp.float32),
          pltpu.VMEM((_CHUNK, d), jnp.float32),
          pltpu.VMEM((_CHUNK, d), jnp.float32),
          pltpu.VMEM((_CHUNK, d), jnp.float32),
          pltpu.SemaphoreType.DMA,
          pltpu.SemaphoreType.DMA,
          pltpu.SemaphoreType.DMA,
          pltpu.SemaphoreType.DMA,
          pltpu.SemaphoreType.DMA,
      ],
  )
  def agg_kernel(x_hbm, src_hbm, dst_hbm, out_hbm, acc, src_ib, dst_ib,
                 rows0, rows1, rows2, rows3, isem, gsem0, gsem1, gsem2,
                 gsem3):
    c = lax.axis_index("c")
    s = lax.axis_index("s")
    wid = s * _NC + c
    blk0 = wid * nblk

    def start_idx(b):
      slot = pl.multiple_of((b % 2) * _IBLK, _IBLK)
      hb = pl.multiple_of((blk0 + b) * _IBLK, _IBLK)
      pltpu.async_copy(src_hbm.at[pl.ds(hb, _IBLK)],
                       src_ib.at[pl.ds(slot, _IBLK)], isem)
      pltpu.async_copy(dst_hbm.at[pl.ds(hb, _IBLK)],
                       dst_ib.at[pl.ds(slot, _IBLK)], isem)

    def wait_idx(b):
      slot = pl.multiple_of((b % 2) * _IBLK, _IBLK)
      hb = pl.multiple_of((blk0 + b) * _IBLK, _IBLK)
      pltpu.make_async_copy(src_hbm.at[pl.ds(hb, _IBLK)],
                            src_ib.at[pl.ds(slot, _IBLK)], isem).wait()
      pltpu.make_async_copy(dst_hbm.at[pl.ds(hb, _IBLK)],
                            dst_ib.at[pl.ds(slot, _IBLK)], isem).wait()

    start_idx(0)

    zero = jnp.zeros((_LANES,), jnp.float32)

    def zrow(r, carry):
      def zcol(j, carry2):
        rows0[r, pl.ds(j * _LANES, _LANES)] = zero
        return carry2

      return lax.fori_loop(0, d // _LANES, zcol, carry)

    lax.fori_loop(0, zr, zrow, 0)

    row0 = s * rows_per_tile

    def zcopy(t, carry):
      pltpu.sync_copy(rows0.at[pl.ds(0, zr)], acc.at[pl.ds(row0 + t * zr, zr)])
      return carry

    lax.fori_loop(0, rows_per_tile // zr, zcopy, 0)
    wait_idx(0)

    rows = (rows0, rows1, rows2, rows3)
    gsems = (gsem0, gsem1, gsem2, gsem3)
    nbuf = len(rows)
    lka = nbuf - 1

    def start_gather(i, j):
      pltpu.async_copy(x_hbm.at[src_ib.at[i % (2 * _IBLK)]], rows[j], gsems[j])

    def wait_gather(i, j):
      pltpu.make_async_copy(x_hbm.at[src_ib.at[i % (2 * _IBLK)]], rows[j],
                            gsems[j]).wait()

    for j in range(lka):
      start_gather(j, j)
    start_idx(1)
    plsc.subcore_barrier()

    def body(t, carry):
      for j in range(nbuf):
        i = t * nbuf + j
        b = i // _IBLK
        r = i % _IBLK

        @pl.when((r == _IBLK - 1 - lka) & (b + 1 < nblk))
        def _():
          wait_idx(b + 1)

        @pl.when(i + lka < cpw)
        def _():
          start_gather(i + lka, (j + lka) % nbuf)

        wait_gather(i, j)
        pltpu.sync_copy(rows[j], acc.at[dst_ib.at[i % (2 * _IBLK)]], add=True)

        @pl.when((r == _IBLK - 1) & (b + 2 < nblk))
        def _():
          start_idx(b + 2)

      return carry

    lax.fori_loop(0, cpw // nbuf, body, 0)
    plsc.subcore_barrier()

    pltpu.sync_copy(
        acc.at[pl.ds(row0, rows_per_tile)],
        out_hbm.at[c, pl.ds(row0, rows_per_tile)],
    )

  return agg_kernel(x, src2, dst2)


def _tc_linear_relu(p0, p1, w_t, b2):
  n_nodes, d = p0.shape
  blk = 1024
  assert n_nodes % blk == 0

  def body(p0_ref, p1_ref, wt_ref, b_ref, o_ref):
    agg = p0_ref[...] + p1_ref[...]
    y = jnp.dot(agg, wt_ref[...], preferred_element_type=jnp.float32)
    o_ref[...] = jnp.maximum(y + b_ref[...], 0.0)

  return pl.pallas_call(
      body,
      grid=(n_nodes // blk,),
      in_specs=[
          pl.BlockSpec((blk, d), lambda i: (i, 0)),
          pl.BlockSpec((blk, d), lambda i: (i, 0)),
          pl.BlockSpec((d, d), lambda i: (0, 0)),
          pl.BlockSpec((1, d), lambda i: (0, 0)),
      ],
      out_specs=pl.BlockSpec((blk, d), lambda i: (i, 0)),
      out_shape=jax.ShapeDtypeStruct((n_nodes, d), jnp.float32),
  )(p0, p1, w_t, b2)


def kernel(x, edge_index, W, b):
  src = edge_index[0].astype(jnp.int32)
  dst = edge_index[1].astype(jnp.int32)
  n_nodes = x.shape[0]
  n_edges = src.shape[0]

  n_pad = n_nodes
  while n_pad % 1024 != 0 or (n_pad // _NS) % 8 != 0:
    n_pad += 1
  pad_rows = n_pad - n_nodes

  unit = _CHUNK * _NW * _IBLK
  e_pad = ((n_edges + unit - 1) // unit) * unit
  extra = e_pad - n_edges
  if extra and pad_rows == 0:
    n_pad += 1
    while n_pad % 1024 != 0 or (n_pad // _NS) % 8 != 0:
      n_pad += 1
    pad_rows = n_pad - n_nodes
  if extra:
    fill = jnp.arange(extra, dtype=jnp.int32)
    src = jnp.concatenate([src, fill % n_nodes])
    dst = jnp.concatenate([dst, n_nodes + fill % max(pad_rows, 1)])
  src2 = src.reshape(-1, _CHUNK)
  dst2 = dst.reshape(-1, _CHUNK)

  partials = _sc_aggregate(x, src2, dst2, n_pad)
  h = _tc_linear_relu(partials[0], partials[1], W.T, b.reshape(1, -1))
  return h[:n_nodes]
